# Initial kernel scaffold; baseline (speedup 1.0000x reference)
import jax
import jax.numpy as jnp
from jax.experimental import pallas as pl


def kernel(x, edge_index, W1, b1, W2, b2):
    raise NotImplementedError("write your pallas kernel here")


# speedup vs baseline: 10.7635x; 10.7635x over previous
import functools

import jax
import jax.numpy as jnp
from jax import lax
from jax.experimental import pallas as pl
from jax.experimental.pallas import tpu as pltpu
from jax.experimental.pallas import tpu_sc as plsc

N = 10000
E = 320000
D_IN = 128
D_H = 128
D_OUT = 64
K = 5
ALPHA = 0.5

NC = 2
NS = 16
NW = NC * NS

N_PAD = 10240
ROWS_PER_TILE = N_PAD // NS
CH = 128
EPW = E // NW
NCH = 80
EPW_PAD = NCH * CH
DUMMY = N

_mesh = plsc.VectorSubcoreMesh(core_axis_name="c", subcore_axis_name="s")
_sc_params = pltpu.CompilerParams(
    needs_layout_passes=False, use_tc_tiling_on_sc=False
)


@functools.partial(
    pl.kernel,
    out_type=jax.ShapeDtypeStruct((NW, N_PAD), jnp.float32),
    mesh=_mesh,
    scratch_types=[
        pltpu.VMEM((N_PAD,), jnp.float32),
        pltpu.VMEM((NCH, CH), jnp.int32),
    ],
    compiler_params=_sc_params,
)
def _degree_hist(dstp_hbm, zeros_hbm, out_hbm, hist, dbuf):
    c = lax.axis_index("c")
    s = lax.axis_index("s")
    wid = s * NC + c
    pltpu.sync_copy(zeros_hbm, hist)
    pltpu.sync_copy(dstp_hbm.at[wid], dbuf)
    ones = jnp.ones((16,), jnp.float32)

    def body(j, carry):
        for cc in range(CH // 16):
            dv = dbuf[j, pl.ds(cc * 16, 16)]
            plsc.addupdate_scatter(hist, [dv], ones)
        return carry

    lax.fori_loop(0, NCH, body, 0)
    pltpu.sync_copy(hist, out_hbm.at[wid])


BM = 512
GRID_M = N_PAD // BM


def _mlp_body(xr, w1r, b1r, w2r, b2r, hr, h0r, g0r, degr):
    h1 = jnp.maximum(
        jnp.dot(xr[...], w1r[...], preferred_element_type=jnp.float32) + b1r[...],
        0.0,
    )
    h = jnp.dot(h1, w2r[...], preferred_element_type=jnp.float32) + b2r[...]
    deg = jnp.sum(hr[...], axis=0) + 1.0
    dinv = lax.rsqrt(deg)[:, None]
    h0r[...] = h
    g0r[...] = h * dinv
    degr[...] = deg[:, None]


_mlp_call = pl.pallas_call(
    _mlp_body,
    grid=(GRID_M,),
    in_specs=[
        pl.BlockSpec((BM, D_IN), lambda i: (i, 0)),
        pl.BlockSpec((D_IN, D_H), lambda i: (0, 0)),
        pl.BlockSpec((1, D_H), lambda i: (0, 0)),
        pl.BlockSpec((D_H, D_OUT), lambda i: (0, 0)),
        pl.BlockSpec((1, D_OUT), lambda i: (0, 0)),
        pl.BlockSpec((NW, BM), lambda i: (0, i)),
    ],
    out_specs=[
        pl.BlockSpec((BM, D_OUT), lambda i: (i, 0)),
        pl.BlockSpec((BM, D_OUT), lambda i: (i, 0)),
        pl.BlockSpec((BM, 1), lambda i: (i, 0)),
    ],
    out_shape=[
        jax.ShapeDtypeStruct((N_PAD, D_OUT), jnp.float32),
        jax.ShapeDtypeStruct((N_PAD, D_OUT), jnp.float32),
        jax.ShapeDtypeStruct((N_PAD, 1), jnp.float32),
    ],
)


@functools.partial(
    pl.kernel,
    out_type=jax.ShapeDtypeStruct((NC, N_PAD, D_OUT), jnp.float32),
    mesh=_mesh,
    scratch_types=[
        pltpu.VMEM_SHARED((N_PAD, D_OUT), jnp.float32),
        pltpu.VMEM((NCH, CH), jnp.int32),
        pltpu.VMEM((NCH, CH), jnp.int32),
        pltpu.VMEM((CH, D_OUT), jnp.float32),
        pltpu.VMEM((CH, D_OUT), jnp.float32),
        pltpu.SemaphoreType.DMA,
    ],
    compiler_params=_sc_params,
)
def _scatter_pass(g_hbm, srcp_hbm, dstp_hbm, zc_hbm, out_hbm,
                  acc, sbuf, dbuf, rows, zbuf, semg):
    c = lax.axis_index("c")
    s = lax.axis_index("s")
    wid = s * NC + c
    tid = s
    pltpu.sync_copy(zc_hbm, zbuf)
    for m in range(ROWS_PER_TILE // CH):
        pltpu.sync_copy(zbuf, acc.at[pl.ds(tid * ROWS_PER_TILE + m * CH, CH)])

    pltpu.sync_copy(srcp_hbm.at[wid], sbuf)
    pltpu.sync_copy(dstp_hbm.at[wid], dbuf)
    plsc.subcore_barrier()

    def body(j, carry):
        pltpu.async_copy(g_hbm.at[sbuf.at[j]], rows, semg).wait()
        pltpu.sync_copy(rows, acc.at[dbuf.at[j]], add=True)
        return carry

    lax.fori_loop(0, NCH, body, 0)
    plsc.subcore_barrier()
    pltpu.sync_copy(
        acc.at[pl.ds(tid * ROWS_PER_TILE, ROWS_PER_TILE)],
        out_hbm.at[c, pl.ds(tid * ROWS_PER_TILE, ROWS_PER_TILE)],
    )


def _combine_body(sr, gr, h0r, degr, hr, gnr):
    dinv = lax.rsqrt(degr[...])
    t = sr[0] + sr[1] + gr[...]
    h = (1.0 - ALPHA) * dinv * t + ALPHA * h0r[...]
    hr[...] = h
    gnr[...] = h * dinv


_combine_call = pl.pallas_call(
    _combine_body,
    grid=(GRID_M,),
    in_specs=[
        pl.BlockSpec((NC, BM, D_OUT), lambda i: (0, i, 0)),
        pl.BlockSpec((BM, D_OUT), lambda i: (i, 0)),
        pl.BlockSpec((BM, D_OUT), lambda i: (i, 0)),
        pl.BlockSpec((BM, 1), lambda i: (i, 0)),
    ],
    out_specs=[
        pl.BlockSpec((BM, D_OUT), lambda i: (i, 0)),
        pl.BlockSpec((BM, D_OUT), lambda i: (i, 0)),
    ],
    out_shape=[
        jax.ShapeDtypeStruct((N_PAD, D_OUT), jnp.float32),
        jax.ShapeDtypeStruct((N_PAD, D_OUT), jnp.float32),
    ],
)


def kernel(x, edge_index, W1, b1, W2, b2):
    src = edge_index[0].reshape(NW, EPW)
    dst = edge_index[1].reshape(NW, EPW)
    pad = ((0, 0), (0, EPW_PAD - EPW))
    srcp = jnp.pad(src, pad, constant_values=0).reshape(NW, NCH, CH)
    dstp = jnp.pad(dst, pad, constant_values=DUMMY).reshape(NW, NCH, CH)
    x_pad = jnp.pad(x, ((0, N_PAD - N), (0, 0)))
    zeros_a = jnp.zeros((N_PAD,), jnp.float32)
    zeros_c = jnp.zeros((CH, D_OUT), jnp.float32)

    hists = _degree_hist(dstp, zeros_a)
    h0, g0, deg = _mlp_call(
        x_pad, W1, b1.reshape(1, D_H), W2, b2.reshape(1, D_OUT), hists
    )
    h, g = h0, g0
    for _ in range(K):
        sp = _scatter_pass(g, srcp, dstp, zeros_c)
        h, g = _combine_call(sp, g, h0, deg)
    return h[:N]


# speedup vs baseline: 12.5510x; 1.1661x over previous
import functools

import jax
import jax.numpy as jnp
from jax import lax
from jax.experimental import pallas as pl
from jax.experimental.pallas import tpu as pltpu
from jax.experimental.pallas import tpu_sc as plsc

N = 10000
E = 320000
D_IN = 128
D_H = 128
D_OUT = 64
K = 5
ALPHA = 0.5

NC = 2
NS = 16
NW = NC * NS

N_PAD = 10240
ROWS_PER_TILE = N_PAD // NS
CH = 128
EPW = E // NW
NCH = 80
EPW_PAD = NCH * CH
DUMMY = N

_mesh = plsc.VectorSubcoreMesh(core_axis_name="c", subcore_axis_name="s")
_sc_params = pltpu.CompilerParams(
    needs_layout_passes=False, use_tc_tiling_on_sc=False
)


@functools.partial(
    pl.kernel,
    out_type=jax.ShapeDtypeStruct((NW, N_PAD), jnp.float32),
    mesh=_mesh,
    scratch_types=[
        pltpu.VMEM((N_PAD,), jnp.float32),
        pltpu.VMEM((NCH, CH), jnp.int32),
    ],
    compiler_params=_sc_params,
)
def _degree_hist(dstp_hbm, zeros_hbm, out_hbm, hist, dbuf):
    c = lax.axis_index("c")
    s = lax.axis_index("s")
    wid = s * NC + c
    pltpu.sync_copy(zeros_hbm, hist)
    pltpu.sync_copy(dstp_hbm.at[wid], dbuf)
    ones = jnp.ones((16,), jnp.float32)

    def body(j, carry):
        for cc in range(CH // 16):
            dv = dbuf[j, pl.ds(cc * 16, 16)]
            plsc.addupdate_scatter(hist, [dv], ones)
        return carry

    lax.fori_loop(0, NCH, body, 0)
    pltpu.sync_copy(hist, out_hbm.at[wid])


BM = 512
GRID_M = N_PAD // BM


def _mlp_body(xr, w1r, b1r, w2r, b2r, hr, h0r, g0r, degr):
    h1 = jnp.maximum(
        jnp.dot(xr[...], w1r[...], preferred_element_type=jnp.float32) + b1r[...],
        0.0,
    )
    h = jnp.dot(h1, w2r[...], preferred_element_type=jnp.float32) + b2r[...]
    deg = jnp.sum(hr[...], axis=0) + 1.0
    dinv = lax.rsqrt(deg)[:, None]
    h0r[...] = h
    g0r[...] = h * dinv
    degr[...] = deg[:, None]


_mlp_call = pl.pallas_call(
    _mlp_body,
    grid=(GRID_M,),
    in_specs=[
        pl.BlockSpec((BM, D_IN), lambda i: (i, 0)),
        pl.BlockSpec((D_IN, D_H), lambda i: (0, 0)),
        pl.BlockSpec((1, D_H), lambda i: (0, 0)),
        pl.BlockSpec((D_H, D_OUT), lambda i: (0, 0)),
        pl.BlockSpec((1, D_OUT), lambda i: (0, 0)),
        pl.BlockSpec((NW, BM), lambda i: (0, i)),
    ],
    out_specs=[
        pl.BlockSpec((BM, D_OUT), lambda i: (i, 0)),
        pl.BlockSpec((BM, D_OUT), lambda i: (i, 0)),
        pl.BlockSpec((BM, 1), lambda i: (i, 0)),
    ],
    out_shape=[
        jax.ShapeDtypeStruct((N_PAD, D_OUT), jnp.float32),
        jax.ShapeDtypeStruct((N_PAD, D_OUT), jnp.float32),
        jax.ShapeDtypeStruct((N_PAD, 1), jnp.float32),
    ],
)


NBUF = 4
NG = NCH // NBUF


@functools.partial(
    pl.kernel,
    out_type=jax.ShapeDtypeStruct((NC, N_PAD, D_OUT), jnp.float32),
    mesh=_mesh,
    scratch_types=[
        pltpu.VMEM_SHARED((N_PAD, D_OUT), jnp.float32),
        pltpu.VMEM((NCH, CH), jnp.int32),
        pltpu.VMEM((NCH, CH), jnp.int32),
        pltpu.VMEM((2, NBUF, CH, D_OUT), jnp.float32),
        pltpu.SemaphoreType.DMA((2,)),
        pltpu.SemaphoreType.DMA((2,)),
        pltpu.SemaphoreType.DMA,
    ],
    compiler_params=_sc_params,
)
def _scatter_pass(g_hbm, srcp_hbm, dstp_hbm, zc_hbm, out_hbm,
                  acc, sbuf, dbuf, rows, semg, sems, semz):
    c = lax.axis_index("c")
    s = lax.axis_index("s")
    wid = s * NC + c
    tid = s

    zcopies = []
    for m in range(ROWS_PER_TILE // CH):
        zcopies.append(pltpu.async_copy(
            zc_hbm, acc.at[pl.ds(tid * ROWS_PER_TILE + m * CH, CH)], semz))
    zcopies.append(pltpu.async_copy(srcp_hbm.at[wid], sbuf, semz))
    zcopies.append(pltpu.async_copy(dstp_hbm.at[wid], dbuf, semz))
    for cp in zcopies:
        cp.wait()
    plsc.subcore_barrier()

    def gathers(g, half):
        for b in range(NBUF):
            pltpu.async_copy(
                g_hbm.at[sbuf.at[g * NBUF + b]], rows.at[half, b],
                semg.at[half])

    def drain_gathers(half):
        for b in range(NBUF):
            pltpu.make_async_copy(
                g_hbm.at[sbuf.at[0]], rows.at[half, b], semg.at[half]).wait()

    def scatters(g, half):
        for b in range(NBUF):
            pltpu.async_copy(
                rows.at[half, b], acc.at[dbuf.at[g * NBUF + b]],
                sems.at[half], add=True)

    def drain_scatters(half):
        for b in range(NBUF):
            pltpu.make_async_copy(
                rows.at[half, b], acc.at[dbuf.at[0]], sems.at[half]).wait()

    gathers(0, 0)

    def pair(p, carry):
        for half in (0, 1):
            other = 1 - half
            g = 2 * p + half
            drain_gathers(half)

            @pl.when(g > 0)
            def _():
                drain_scatters(other)

            @pl.when(g + 1 < NG)
            def _():
                gathers(g + 1, other)

            scatters(g, half)
        return carry

    lax.fori_loop(0, NG // 2, pair, 0)
    drain_scatters(1)
    plsc.subcore_barrier()
    pltpu.sync_copy(
        acc.at[pl.ds(tid * ROWS_PER_TILE, ROWS_PER_TILE)],
        out_hbm.at[c, pl.ds(tid * ROWS_PER_TILE, ROWS_PER_TILE)],
    )


def _combine_body(sr, gr, h0r, degr, hr, gnr):
    dinv = lax.rsqrt(degr[...])
    t = sr[0] + sr[1] + gr[...]
    h = (1.0 - ALPHA) * dinv * t + ALPHA * h0r[...]
    hr[...] = h
    gnr[...] = h * dinv


_combine_call = pl.pallas_call(
    _combine_body,
    grid=(GRID_M,),
    in_specs=[
        pl.BlockSpec((NC, BM, D_OUT), lambda i: (0, i, 0)),
        pl.BlockSpec((BM, D_OUT), lambda i: (i, 0)),
        pl.BlockSpec((BM, D_OUT), lambda i: (i, 0)),
        pl.BlockSpec((BM, 1), lambda i: (i, 0)),
    ],
    out_specs=[
        pl.BlockSpec((BM, D_OUT), lambda i: (i, 0)),
        pl.BlockSpec((BM, D_OUT), lambda i: (i, 0)),
    ],
    out_shape=[
        jax.ShapeDtypeStruct((N_PAD, D_OUT), jnp.float32),
        jax.ShapeDtypeStruct((N_PAD, D_OUT), jnp.float32),
    ],
)


def kernel(x, edge_index, W1, b1, W2, b2):
    src = edge_index[0].reshape(NW, EPW)
    dst = edge_index[1].reshape(NW, EPW)
    pad = ((0, 0), (0, EPW_PAD - EPW))
    srcp = jnp.pad(src, pad, constant_values=0).reshape(NW, NCH, CH)
    dstp = jnp.pad(dst, pad, constant_values=DUMMY).reshape(NW, NCH, CH)
    x_pad = jnp.pad(x, ((0, N_PAD - N), (0, 0)))
    zeros_a = jnp.zeros((N_PAD,), jnp.float32)
    zeros_c = jnp.zeros((CH, D_OUT), jnp.float32)

    hists = _degree_hist(dstp, zeros_a)
    h0, g0, deg = _mlp_call(
        x_pad, W1, b1.reshape(1, D_H), W2, b2.reshape(1, D_OUT), hists
    )
    h, g = h0, g0
    for _ in range(K):
        sp = _scatter_pass(g, srcp, dstp, zeros_c)
        h, g = _combine_call(sp, g, h0, deg)
    return h[:N]


# speedup vs baseline: 24.4066x; 1.9446x over previous
import functools

import jax
import jax.numpy as jnp
from jax import lax
from jax.experimental import pallas as pl
from jax.experimental.pallas import tpu as pltpu
from jax.experimental.pallas import tpu_sc as plsc

N = 10000
E = 320000
D_IN = 128
D_H = 128
D_OUT = 64
K = 5
ALPHA = 0.5

NC = 2
NS = 16
NW = NC * NS

N_PAD = 10240
ROWS_PER_TILE = N_PAD // NS
CH = 128
EPW = E // NW
NCH = 80
EPW_PAD = NCH * CH
DUMMY = N

_mesh = plsc.VectorSubcoreMesh(core_axis_name="c", subcore_axis_name="s")
_sc_params = pltpu.CompilerParams(
    needs_layout_passes=False, use_tc_tiling_on_sc=False
)


@functools.partial(
    pl.kernel,
    out_type=jax.ShapeDtypeStruct((NW, N_PAD), jnp.float32),
    mesh=_mesh,
    scratch_types=[
        pltpu.VMEM((N_PAD,), jnp.float32),
        pltpu.VMEM((NCH, CH), jnp.int32),
    ],
    compiler_params=_sc_params,
)
def _degree_hist(dstp_hbm, zeros_hbm, out_hbm, hist, dbuf):
    c = lax.axis_index("c")
    s = lax.axis_index("s")
    wid = s * NC + c
    pltpu.sync_copy(zeros_hbm, hist)
    pltpu.sync_copy(dstp_hbm.at[wid], dbuf)
    ones = jnp.ones((16,), jnp.float32)

    def body(j, carry):
        for cc in range(CH // 16):
            dv = dbuf[j, pl.ds(cc * 16, 16)]
            plsc.addupdate_scatter(hist, [dv], ones)
        return carry

    lax.fori_loop(0, NCH, body, 0)
    pltpu.sync_copy(hist, out_hbm.at[wid])


BM = 512
GRID_M = N_PAD // BM


def _mlp_body(xr, w1r, b1r, w2r, b2r, hr, h0r, g0r, degr):
    h1 = jnp.maximum(
        jnp.dot(xr[...], w1r[...], preferred_element_type=jnp.float32) + b1r[...],
        0.0,
    )
    h = jnp.dot(h1, w2r[...], preferred_element_type=jnp.float32) + b2r[...]
    deg = jnp.sum(hr[...], axis=0) + 1.0
    dinv = lax.rsqrt(deg)[:, None]
    h0r[...] = h
    g0r[...] = h * dinv
    degr[...] = deg[:, None]


_mlp_call = pl.pallas_call(
    _mlp_body,
    grid=(GRID_M,),
    in_specs=[
        pl.BlockSpec((BM, D_IN), lambda i: (i, 0)),
        pl.BlockSpec((D_IN, D_H), lambda i: (0, 0)),
        pl.BlockSpec((1, D_H), lambda i: (0, 0)),
        pl.BlockSpec((D_H, D_OUT), lambda i: (0, 0)),
        pl.BlockSpec((1, D_OUT), lambda i: (0, 0)),
        pl.BlockSpec((NW, BM), lambda i: (0, i)),
    ],
    out_specs=[
        pl.BlockSpec((BM, D_OUT), lambda i: (i, 0)),
        pl.BlockSpec((BM, D_OUT), lambda i: (i, 0)),
        pl.BlockSpec((BM, 1), lambda i: (i, 0)),
    ],
    out_shape=[
        jax.ShapeDtypeStruct((N_PAD, D_OUT), jnp.float32),
        jax.ShapeDtypeStruct((N_PAD, D_OUT), jnp.float32),
        jax.ShapeDtypeStruct((N_PAD, 1), jnp.float32),
    ],
)


NBUF = 2
NG = NCH // NBUF


@functools.partial(
    pl.kernel,
    out_type=jax.ShapeDtypeStruct((NC, N_PAD, D_OUT), jnp.float32),
    mesh=_mesh,
    scratch_types=[
        pltpu.VMEM_SHARED((N_PAD, D_OUT), jnp.float32),
        pltpu.VMEM_SHARED((N_PAD, D_OUT), jnp.float32),
        pltpu.VMEM((NCH, CH), jnp.int32),
        pltpu.VMEM((2, NBUF, CH), jnp.int32),
        pltpu.VMEM((2, NBUF, CH, D_OUT), jnp.float32),
        pltpu.SemaphoreType.DMA((2,)),
        pltpu.SemaphoreType.DMA((2,)),
        pltpu.SemaphoreType.DMA((2,)),
        pltpu.SemaphoreType.DMA,
    ],
    compiler_params=_sc_params,
)
def _scatter_pass(g_hbm, srcp_hbm, dstp_hbm, zc_hbm, out_hbm,
                  acc, gsp, dbuf, sidx, rows, semg, sems, semi, semz):
    c = lax.axis_index("c")
    s = lax.axis_index("s")
    wid = s * NC + c
    tid = s

    zcopies = []
    for m in range(ROWS_PER_TILE // CH):
        zcopies.append(pltpu.async_copy(
            zc_hbm, acc.at[pl.ds(tid * ROWS_PER_TILE + m * CH, CH)], semz))
    zcopies.append(pltpu.async_copy(
        g_hbm.at[pl.ds(tid * ROWS_PER_TILE, ROWS_PER_TILE)],
        gsp.at[pl.ds(tid * ROWS_PER_TILE, ROWS_PER_TILE)], semz))
    zcopies.append(pltpu.async_copy(dstp_hbm.at[wid], dbuf, semz))
    zcopies.append(pltpu.async_copy(
        srcp_hbm.at[wid, pl.ds(0, NBUF)], sidx.at[0], semz))
    for cp in zcopies:
        cp.wait()
    plsc.subcore_barrier()

    def load_sidx(g, half):
        pltpu.async_copy(
            srcp_hbm.at[wid, pl.ds(g * NBUF, NBUF)], sidx.at[half],
            semi.at[half])

    def drain_sidx(half):
        pltpu.make_async_copy(
            srcp_hbm.at[wid, pl.ds(0, NBUF)], sidx.at[half],
            semi.at[half]).wait()

    def gathers(g, half):
        for b in range(NBUF):
            pltpu.async_copy(
                gsp.at[sidx.at[half, b]], rows.at[half, b], semg.at[half])

    def drain_gathers(half):
        for b in range(NBUF):
            pltpu.make_async_copy(
                gsp.at[sidx.at[0, 0]], rows.at[half, b], semg.at[half]).wait()

    def scatters(g, half):
        for b in range(NBUF):
            pltpu.async_copy(
                rows.at[half, b], acc.at[dbuf.at[g * NBUF + b]],
                sems.at[half], add=True)

    def drain_scatters(half):
        for b in range(NBUF):
            pltpu.make_async_copy(
                rows.at[half, b], acc.at[dbuf.at[0]], sems.at[half]).wait()

    gathers(0, 0)
    load_sidx(1, 1)

    def pair(p, carry):
        for half in (0, 1):
            other = 1 - half
            g = 2 * p + half
            drain_gathers(half)

            @pl.when(g + 2 < NG)
            def _():
                load_sidx(g + 2, half)

            @pl.when(g > 0)
            def _():
                drain_scatters(other)

            @pl.when(g + 1 < NG)
            def _():
                drain_sidx(other)
                gathers(g + 1, other)

            scatters(g, half)
        return carry

    lax.fori_loop(0, NG // 2, pair, 0)
    drain_scatters(1)
    plsc.subcore_barrier()
    pltpu.sync_copy(
        acc.at[pl.ds(tid * ROWS_PER_TILE, ROWS_PER_TILE)],
        out_hbm.at[c, pl.ds(tid * ROWS_PER_TILE, ROWS_PER_TILE)],
    )


def _combine_body(sr, gr, h0r, degr, hr, gnr):
    dinv = lax.rsqrt(degr[...])
    t = sr[0] + sr[1] + gr[...]
    h = (1.0 - ALPHA) * dinv * t + ALPHA * h0r[...]
    hr[...] = h
    gnr[...] = h * dinv


_combine_call = pl.pallas_call(
    _combine_body,
    grid=(GRID_M,),
    in_specs=[
        pl.BlockSpec((NC, BM, D_OUT), lambda i: (0, i, 0)),
        pl.BlockSpec((BM, D_OUT), lambda i: (i, 0)),
        pl.BlockSpec((BM, D_OUT), lambda i: (i, 0)),
        pl.BlockSpec((BM, 1), lambda i: (i, 0)),
    ],
    out_specs=[
        pl.BlockSpec((BM, D_OUT), lambda i: (i, 0)),
        pl.BlockSpec((BM, D_OUT), lambda i: (i, 0)),
    ],
    out_shape=[
        jax.ShapeDtypeStruct((N_PAD, D_OUT), jnp.float32),
        jax.ShapeDtypeStruct((N_PAD, D_OUT), jnp.float32),
    ],
)


def kernel(x, edge_index, W1, b1, W2, b2):
    src = edge_index[0].reshape(NW, EPW)
    dst = edge_index[1].reshape(NW, EPW)
    pad = ((0, 0), (0, EPW_PAD - EPW))
    srcp = jnp.pad(src, pad, constant_values=0).reshape(NW, NCH, CH)
    dstp = jnp.pad(dst, pad, constant_values=DUMMY).reshape(NW, NCH, CH)
    x_pad = jnp.pad(x, ((0, N_PAD - N), (0, 0)))
    zeros_a = jnp.zeros((N_PAD,), jnp.float32)
    zeros_c = jnp.zeros((CH, D_OUT), jnp.float32)

    hists = _degree_hist(dstp, zeros_a)
    h0, g0, deg = _mlp_call(
        x_pad, W1, b1.reshape(1, D_H), W2, b2.reshape(1, D_OUT), hists
    )
    h, g = h0, g0
    for _ in range(K):
        sp = _scatter_pass(g, srcp, dstp, zeros_c)
        h, g = _combine_call(sp, g, h0, deg)
    return h[:N]


# speedup vs baseline: 25.0816x; 1.0277x over previous
import functools

import jax
import jax.numpy as jnp
from jax import lax
from jax.experimental import pallas as pl
from jax.experimental.pallas import tpu as pltpu
from jax.experimental.pallas import tpu_sc as plsc

N = 10000
E = 320000
D_IN = 128
D_H = 128
D_OUT = 64
K = 5
ALPHA = 0.5

NC = 2
NS = 16
NW = NC * NS

N_PAD = 10240
ROWS_PER_TILE = N_PAD // NS
CH = 112
EPW = E // NW
NCH = 90
EPW_PAD = NCH * CH
DUMMY = N

_mesh = plsc.VectorSubcoreMesh(core_axis_name="c", subcore_axis_name="s")
_sc_params = pltpu.CompilerParams(
    needs_layout_passes=False, use_tc_tiling_on_sc=False
)


@functools.partial(
    pl.kernel,
    out_type=jax.ShapeDtypeStruct((NW, N_PAD), jnp.float32),
    mesh=_mesh,
    scratch_types=[
        pltpu.VMEM((N_PAD,), jnp.float32),
        pltpu.VMEM((NCH, CH), jnp.int32),
    ],
    compiler_params=_sc_params,
)
def _degree_hist(dstp_hbm, zeros_hbm, out_hbm, hist, dbuf):
    c = lax.axis_index("c")
    s = lax.axis_index("s")
    wid = s * NC + c
    pltpu.sync_copy(zeros_hbm, hist)
    pltpu.sync_copy(dstp_hbm.at[wid], dbuf)
    ones = jnp.ones((16,), jnp.float32)

    def body(j, carry):
        for cc in range(CH // 16):
            dv = dbuf[j, pl.ds(cc * 16, 16)]
            plsc.addupdate_scatter(hist, [dv], ones)
        return carry

    lax.fori_loop(0, NCH, body, 0)
    pltpu.sync_copy(hist, out_hbm.at[wid])


BM = 512
GRID_M = N_PAD // BM


def _mlp_body(xr, w1r, b1r, w2r, b2r, hr, h0r, g0r, degr):
    h1 = jnp.maximum(
        jnp.dot(xr[...], w1r[...], preferred_element_type=jnp.float32) + b1r[...],
        0.0,
    )
    h = jnp.dot(h1, w2r[...], preferred_element_type=jnp.float32) + b2r[...]
    deg = jnp.sum(hr[...], axis=0) + 1.0
    dinv = lax.rsqrt(deg)[:, None]
    h0r[...] = h
    g0r[...] = h * dinv
    degr[...] = deg[:, None]


_mlp_call = pl.pallas_call(
    _mlp_body,
    grid=(GRID_M,),
    in_specs=[
        pl.BlockSpec((BM, D_IN), lambda i: (i, 0)),
        pl.BlockSpec((D_IN, D_H), lambda i: (0, 0)),
        pl.BlockSpec((1, D_H), lambda i: (0, 0)),
        pl.BlockSpec((D_H, D_OUT), lambda i: (0, 0)),
        pl.BlockSpec((1, D_OUT), lambda i: (0, 0)),
        pl.BlockSpec((NW, BM), lambda i: (0, i)),
    ],
    out_specs=[
        pl.BlockSpec((BM, D_OUT), lambda i: (i, 0)),
        pl.BlockSpec((BM, D_OUT), lambda i: (i, 0)),
        pl.BlockSpec((BM, 1), lambda i: (i, 0)),
    ],
    out_shape=[
        jax.ShapeDtypeStruct((N_PAD, D_OUT), jnp.float32),
        jax.ShapeDtypeStruct((N_PAD, D_OUT), jnp.float32),
        jax.ShapeDtypeStruct((N_PAD, 1), jnp.float32),
    ],
)


NBUF = 3
NG = NCH // NBUF
ZCH = 128


@functools.partial(
    pl.kernel,
    out_type=jax.ShapeDtypeStruct((NC, N_PAD, D_OUT), jnp.float32),
    mesh=_mesh,
    scratch_types=[
        pltpu.VMEM_SHARED((N_PAD, D_OUT), jnp.float32),
        pltpu.VMEM_SHARED((N_PAD, D_OUT), jnp.float32),
        pltpu.VMEM((2, NBUF, CH), jnp.int32),
        pltpu.VMEM((2, NBUF, CH), jnp.int32),
        pltpu.VMEM((2, NBUF, CH, D_OUT), jnp.float32),
        pltpu.SemaphoreType.DMA((2,)),
        pltpu.SemaphoreType.DMA((2,)),
        pltpu.SemaphoreType.DMA((2,)),
        pltpu.SemaphoreType.DMA((2,)),
        pltpu.SemaphoreType.DMA,
    ],
    compiler_params=_sc_params,
)
def _scatter_pass(g_hbm, srcp_hbm, dstp_hbm, zc_hbm, out_hbm,
                  acc, gsp, sidx, didx, rows, semg, sems, semsi, semdi, semz):
    c = lax.axis_index("c")
    s = lax.axis_index("s")
    wid = s * NC + c
    tid = s

    zcopies = []
    for m in range(ROWS_PER_TILE // ZCH):
        zcopies.append(pltpu.async_copy(
            zc_hbm, acc.at[pl.ds(tid * ROWS_PER_TILE + m * ZCH, ZCH)], semz))
    zcopies.append(pltpu.async_copy(
        g_hbm.at[pl.ds(tid * ROWS_PER_TILE, ROWS_PER_TILE)],
        gsp.at[pl.ds(tid * ROWS_PER_TILE, ROWS_PER_TILE)], semz))

    def load_sidx(g, slot):
        pltpu.async_copy(
            srcp_hbm.at[wid, pl.ds(g * NBUF, NBUF)], sidx.at[slot],
            semsi.at[slot])

    def drain_sidx(slot):
        pltpu.make_async_copy(
            srcp_hbm.at[wid, pl.ds(0, NBUF)], sidx.at[slot],
            semsi.at[slot]).wait()

    def load_didx(g, slot):
        pltpu.async_copy(
            dstp_hbm.at[wid, pl.ds(g * NBUF, NBUF)], didx.at[slot],
            semdi.at[slot])

    def drain_didx(slot):
        pltpu.make_async_copy(
            dstp_hbm.at[wid, pl.ds(0, NBUF)], didx.at[slot],
            semdi.at[slot]).wait()

    def gathers(half):
        for b in range(NBUF):
            pltpu.async_copy(
                gsp.at[sidx.at[half, b]], rows.at[half, b], semg.at[half])

    def drain_gathers(half):
        for b in range(NBUF):
            pltpu.make_async_copy(
                gsp.at[sidx.at[0, 0]], rows.at[half, b], semg.at[half]).wait()

    def scatters(half):
        for b in range(NBUF):
            pltpu.async_copy(
                rows.at[half, b], acc.at[didx.at[half, b]],
                sems.at[half], add=True)

    def drain_scatters(half):
        for b in range(NBUF):
            pltpu.make_async_copy(
                rows.at[half, b], acc.at[didx.at[0, 0]], sems.at[half]).wait()

    load_sidx(0, 0)
    load_didx(0, 0)
    for cp in zcopies:
        cp.wait()
    drain_sidx(0)
    plsc.subcore_barrier()
    gathers(0)
    load_sidx(1, 1)

    def pair(p, carry):
        for half in (0, 1):
            other = 1 - half
            g = 2 * p + half
            drain_gathers(half)

            @pl.when(g + 2 < NG)
            def _():
                load_sidx(g + 2, half)

            @pl.when(g > 0)
            def _():
                drain_scatters(other)

            @pl.when(g + 1 < NG)
            def _():
                load_didx(g + 1, other)
                drain_sidx(other)
                gathers(other)

            drain_didx(half)
            scatters(half)
        return carry

    lax.fori_loop(0, NG // 2, pair, 0)
    drain_scatters(1)
    plsc.subcore_barrier()
    pltpu.sync_copy(
        acc.at[pl.ds(tid * ROWS_PER_TILE, ROWS_PER_TILE)],
        out_hbm.at[c, pl.ds(tid * ROWS_PER_TILE, ROWS_PER_TILE)],
    )


def _combine_body(sr, gr, h0r, degr, hr, gnr):
    dinv = lax.rsqrt(degr[...])
    t = sr[0] + sr[1] + gr[...]
    h = (1.0 - ALPHA) * dinv * t + ALPHA * h0r[...]
    hr[...] = h
    gnr[...] = h * dinv


_combine_call = pl.pallas_call(
    _combine_body,
    grid=(GRID_M,),
    in_specs=[
        pl.BlockSpec((NC, BM, D_OUT), lambda i: (0, i, 0)),
        pl.BlockSpec((BM, D_OUT), lambda i: (i, 0)),
        pl.BlockSpec((BM, D_OUT), lambda i: (i, 0)),
        pl.BlockSpec((BM, 1), lambda i: (i, 0)),
    ],
    out_specs=[
        pl.BlockSpec((BM, D_OUT), lambda i: (i, 0)),
        pl.BlockSpec((BM, D_OUT), lambda i: (i, 0)),
    ],
    out_shape=[
        jax.ShapeDtypeStruct((N_PAD, D_OUT), jnp.float32),
        jax.ShapeDtypeStruct((N_PAD, D_OUT), jnp.float32),
    ],
)


def kernel(x, edge_index, W1, b1, W2, b2):
    src = edge_index[0].reshape(NW, EPW)
    dst = edge_index[1].reshape(NW, EPW)
    pad = ((0, 0), (0, EPW_PAD - EPW))
    srcp = jnp.pad(src, pad, constant_values=0).reshape(NW, NCH, CH)
    dstp = jnp.pad(dst, pad, constant_values=DUMMY).reshape(NW, NCH, CH)
    x_pad = jnp.pad(x, ((0, N_PAD - N), (0, 0)))
    zeros_a = jnp.zeros((N_PAD,), jnp.float32)
    zeros_c = jnp.zeros((ZCH, D_OUT), jnp.float32)

    hists = _degree_hist(dstp, zeros_a)
    h0, g0, deg = _mlp_call(
        x_pad, W1, b1.reshape(1, D_H), W2, b2.reshape(1, D_OUT), hists
    )
    h, g = h0, g0
    for _ in range(K):
        sp = _scatter_pass(g, srcp, dstp, zeros_c)
        h, g = _combine_call(sp, g, h0, deg)
    return h[:N]


# speedup vs baseline: 30.0885x; 1.1996x over previous
import functools

import jax
import jax.numpy as jnp
from jax import lax
from jax.experimental import pallas as pl
from jax.experimental.pallas import tpu as pltpu
from jax.experimental.pallas import tpu_sc as plsc

N = 10000
E = 320000
D_IN = 128
D_H = 128
D_OUT = 64
K = 5
ALPHA = 0.5

NC = 2
NS = 16
NW = NC * NS

N_PAD = 10240
ROWS_PER_TILE = N_PAD // NS
CH = 112
EPW = E // NW
NCH = 90
EPW_PAD = NCH * CH
DUMMY = N

_mesh = plsc.VectorSubcoreMesh(core_axis_name="c", subcore_axis_name="s")
_sc_params = pltpu.CompilerParams(
    needs_layout_passes=False, use_tc_tiling_on_sc=False
)


@functools.partial(
    pl.kernel,
    out_type=jax.ShapeDtypeStruct((NW, N_PAD), jnp.float32),
    mesh=_mesh,
    scratch_types=[
        pltpu.VMEM((N_PAD,), jnp.float32),
        pltpu.VMEM((NCH, CH), jnp.int32),
    ],
    compiler_params=_sc_params,
)
def _degree_hist(dstp_hbm, zeros_hbm, out_hbm, hist, dbuf):
    c = lax.axis_index("c")
    s = lax.axis_index("s")
    wid = s * NC + c
    pltpu.sync_copy(zeros_hbm, hist)
    pltpu.sync_copy(dstp_hbm.at[wid], dbuf)
    ones = jnp.ones((16,), jnp.float32)

    def body(j, carry):
        for cc in range(CH // 16):
            dv = dbuf[j, pl.ds(cc * 16, 16)]
            plsc.addupdate_scatter(hist, [dv], ones)
        return carry

    lax.fori_loop(0, NCH, body, 0)
    pltpu.sync_copy(hist, out_hbm.at[wid])


BM = 512
GRID_M = N_PAD // BM


def _mlp_body(xr, w1r, b1r, w2r, b2r, hr, g0r, ar, br, sdr):
    h1 = jnp.maximum(
        jnp.dot(xr[...], w1r[...], preferred_element_type=jnp.float32) + b1r[...],
        0.0,
    )
    h = jnp.dot(h1, w2r[...], preferred_element_type=jnp.float32) + b2r[...]
    deg = jnp.sum(hr[...], axis=0) + 1.0
    dinv = lax.rsqrt(deg)[:, None]
    g0 = h * dinv
    g0r[...] = g0
    ar[...] = jnp.broadcast_to((1.0 - ALPHA) * dinv * dinv, (BM, D_OUT))
    br[...] = ALPHA * g0
    sdr[...] = jnp.broadcast_to(jnp.sqrt(deg)[:, None], (BM, D_OUT))


_mlp_call = pl.pallas_call(
    _mlp_body,
    grid=(GRID_M,),
    in_specs=[
        pl.BlockSpec((BM, D_IN), lambda i: (i, 0)),
        pl.BlockSpec((D_IN, D_H), lambda i: (0, 0)),
        pl.BlockSpec((1, D_H), lambda i: (0, 0)),
        pl.BlockSpec((D_H, D_OUT), lambda i: (0, 0)),
        pl.BlockSpec((1, D_OUT), lambda i: (0, 0)),
        pl.BlockSpec((NW, BM), lambda i: (0, i)),
    ],
    out_specs=[
        pl.BlockSpec((BM, D_OUT), lambda i: (i, 0)),
        pl.BlockSpec((BM, D_OUT), lambda i: (i, 0)),
        pl.BlockSpec((BM, D_OUT), lambda i: (i, 0)),
        pl.BlockSpec((BM, D_OUT), lambda i: (i, 0)),
    ],
    out_shape=[
        jax.ShapeDtypeStruct((N_PAD, D_OUT), jnp.float32),
        jax.ShapeDtypeStruct((N_PAD, D_OUT), jnp.float32),
        jax.ShapeDtypeStruct((N_PAD, D_OUT), jnp.float32),
        jax.ShapeDtypeStruct((N_PAD, D_OUT), jnp.float32),
    ],
)


NBUF = 3
NG = NCH // NBUF
ZCH = 128


@functools.partial(
    pl.kernel,
    out_type=jax.ShapeDtypeStruct((NC, N_PAD, D_OUT), jnp.float32),
    mesh=_mesh,
    scratch_types=[
        pltpu.VMEM_SHARED((N_PAD, D_OUT), jnp.float32),
        pltpu.VMEM_SHARED((N_PAD, D_OUT), jnp.float32),
        pltpu.VMEM((2, NBUF, CH), jnp.int32),
        pltpu.VMEM((2, NBUF, CH), jnp.int32),
        pltpu.VMEM((2, NBUF, CH, D_OUT), jnp.float32),
        pltpu.SemaphoreType.DMA((2,)),
        pltpu.SemaphoreType.DMA((2,)),
        pltpu.SemaphoreType.DMA((2,)),
        pltpu.SemaphoreType.DMA((2,)),
        pltpu.SemaphoreType.DMA,
    ],
    compiler_params=_sc_params,
)
def _scatter_pass(g_hbm, srcp_hbm, dstp_hbm, zc_hbm, out_hbm,
                  acc, gsp, sidx, didx, rows, semg, sems, semsi, semdi, semz):
    c = lax.axis_index("c")
    s = lax.axis_index("s")
    wid = s * NC + c
    tid = s

    @pl.when(c == 0)
    def _():
        for m in range(ROWS_PER_TILE // ZCH):
            r0 = tid * ROWS_PER_TILE + m * ZCH
            pltpu.async_copy(
                g_hbm.at[pl.ds(r0, ZCH)], acc.at[pl.ds(r0, ZCH)], semz)

    @pl.when(c != 0)
    def _():
        for m in range(ROWS_PER_TILE // ZCH):
            r0 = tid * ROWS_PER_TILE + m * ZCH
            pltpu.async_copy(zc_hbm, acc.at[pl.ds(r0, ZCH)], semz)

    pltpu.async_copy(
        g_hbm.at[pl.ds(tid * ROWS_PER_TILE, ROWS_PER_TILE)],
        gsp.at[pl.ds(tid * ROWS_PER_TILE, ROWS_PER_TILE)], semz)
    zcopies = []
    for m in range(ROWS_PER_TILE // ZCH):
        zcopies.append(pltpu.make_async_copy(
            zc_hbm, acc.at[pl.ds(tid * ROWS_PER_TILE + m * ZCH, ZCH)], semz))
    zcopies.append(pltpu.make_async_copy(
        g_hbm.at[pl.ds(tid * ROWS_PER_TILE, ROWS_PER_TILE)],
        gsp.at[pl.ds(tid * ROWS_PER_TILE, ROWS_PER_TILE)], semz))

    def load_sidx(g, slot):
        pltpu.async_copy(
            srcp_hbm.at[wid, pl.ds(g * NBUF, NBUF)], sidx.at[slot],
            semsi.at[slot])

    def drain_sidx(slot):
        pltpu.make_async_copy(
            srcp_hbm.at[wid, pl.ds(0, NBUF)], sidx.at[slot],
            semsi.at[slot]).wait()

    def load_didx(g, slot):
        pltpu.async_copy(
            dstp_hbm.at[wid, pl.ds(g * NBUF, NBUF)], didx.at[slot],
            semdi.at[slot])

    def drain_didx(slot):
        pltpu.make_async_copy(
            dstp_hbm.at[wid, pl.ds(0, NBUF)], didx.at[slot],
            semdi.at[slot]).wait()

    def gathers(half):
        for b in range(NBUF):
            pltpu.async_copy(
                gsp.at[sidx.at[half, b]], rows.at[half, b], semg.at[half])

    def drain_gathers(half):
        for b in range(NBUF):
            pltpu.make_async_copy(
                gsp.at[sidx.at[0, 0]], rows.at[half, b], semg.at[half]).wait()

    def scatters(half):
        for b in range(NBUF):
            pltpu.async_copy(
                rows.at[half, b], acc.at[didx.at[half, b]],
                sems.at[half], add=True)

    def drain_scatters(half):
        for b in range(NBUF):
            pltpu.make_async_copy(
                rows.at[half, b], acc.at[didx.at[0, 0]], sems.at[half]).wait()

    load_sidx(0, 0)
    load_didx(0, 0)
    for cp in zcopies:
        cp.wait()
    drain_sidx(0)
    plsc.subcore_barrier()
    gathers(0)
    load_sidx(1, 1)

    def pair(p, carry):
        for half in (0, 1):
            other = 1 - half
            g = 2 * p + half
            drain_gathers(half)

            @pl.when(g + 2 < NG)
            def _():
                load_sidx(g + 2, half)

            @pl.when(g > 0)
            def _():
                drain_scatters(other)

            @pl.when(g + 1 < NG)
            def _():
                load_didx(g + 1, other)
                drain_sidx(other)
                gathers(other)

            drain_didx(half)
            scatters(half)
        return carry

    lax.fori_loop(0, NG // 2, pair, 0)
    drain_scatters(1)
    plsc.subcore_barrier()
    pltpu.sync_copy(
        acc.at[pl.ds(tid * ROWS_PER_TILE, ROWS_PER_TILE)],
        out_hbm.at[c, pl.ds(tid * ROWS_PER_TILE, ROWS_PER_TILE)],
    )


RPW = N_PAD // NW


@functools.partial(
    pl.kernel,
    out_type=[
        jax.ShapeDtypeStruct((N_PAD, D_OUT), jnp.float32),
        jax.ShapeDtypeStruct((N_PAD, D_OUT), jnp.float32),
    ],
    mesh=_mesh,
    scratch_types=[
        pltpu.VMEM((RPW, D_OUT), jnp.float32),
        pltpu.VMEM((RPW, D_OUT), jnp.float32),
        pltpu.VMEM((RPW, D_OUT), jnp.float32),
        pltpu.VMEM((RPW, D_OUT), jnp.float32),
        pltpu.VMEM((RPW, D_OUT), jnp.float32),
        pltpu.SemaphoreType.DMA,
    ],
    compiler_params=_sc_params,
)
def _combine_pass(s_hbm, a_hbm, b_hbm, sd_hbm, g_hbm, h_hbm,
                  b0, b1, ba, bb, bs, sem):
    c = lax.axis_index("c")
    s = lax.axis_index("s")
    wid = s * NC + c
    r0 = wid * RPW
    cps = [
        pltpu.async_copy(s_hbm.at[0, pl.ds(r0, RPW)], b0, sem),
        pltpu.async_copy(s_hbm.at[1, pl.ds(r0, RPW)], b1, sem),
        pltpu.async_copy(a_hbm.at[pl.ds(r0, RPW)], ba, sem),
        pltpu.async_copy(b_hbm.at[pl.ds(r0, RPW)], bb, sem),
        pltpu.async_copy(sd_hbm.at[pl.ds(r0, RPW)], bs, sem),
    ]
    for cp in cps:
        cp.wait()

    def row(r, carry):
        for j in range(D_OUT // 16):
            cs = pl.ds(j * 16, 16)
            g = (b0[r, cs] + b1[r, cs]) * ba[r, cs] + bb[r, cs]
            h = g * bs[r, cs]
            b0[r, cs] = g
            b1[r, cs] = h
        return carry

    lax.fori_loop(0, RPW, row, 0)
    pltpu.sync_copy(b0, g_hbm.at[pl.ds(r0, RPW)])
    pltpu.sync_copy(b1, h_hbm.at[pl.ds(r0, RPW)])


def kernel(x, edge_index, W1, b1, W2, b2):
    src = edge_index[0].reshape(NW, EPW)
    dst = edge_index[1].reshape(NW, EPW)
    pad = ((0, 0), (0, EPW_PAD - EPW))
    srcp = jnp.pad(src, pad, constant_values=0).reshape(NW, NCH, CH)
    dstp = jnp.pad(dst, pad, constant_values=DUMMY).reshape(NW, NCH, CH)
    x_pad = jnp.pad(x, ((0, N_PAD - N), (0, 0)))
    zeros_a = jnp.zeros((N_PAD,), jnp.float32)
    zeros_c = jnp.zeros((ZCH, D_OUT), jnp.float32)

    hists = _degree_hist(dstp, zeros_a)
    g, aarr, barr, sdeg = _mlp_call(
        x_pad, W1, b1.reshape(1, D_H), W2, b2.reshape(1, D_OUT), hists
    )
    h = g
    for _ in range(K):
        sp = _scatter_pass(g, srcp, dstp, zeros_c)
        g, h = _combine_pass(sp, aarr, barr, sdeg)
    return h[:N]
